# Initial kernel scaffold; baseline (speedup 1.0000x reference)
import jax
import jax.numpy as jnp
from jax.experimental import pallas as pl


def kernel(pred0, pred1, pred2, tbox0, tbox1, tbox2, anch0, anch1, anch2, b0, a0, gj0, gi0, tcls0, b1, a1, gj1, gi1, tcls1, b2, a2, gj2, gi2, tcls2):
    raise NotImplementedError("write your pallas kernel here")


# speedup vs baseline: 1.4982x; 1.4982x over previous
import functools
import math

import jax
import jax.numpy as jnp
from jax import lax
from jax.experimental import pallas as pl
from jax.experimental.pallas import tpu as pltpu
from jax.experimental.pallas import tpu_sc as plsc

_HW = (20, 40, 80)
_N = 300
_C = 85
_NCLS = 80
_BAL = (0.4, 1.0, 4.0)
_CHUNK = 128
_NW = 32
_CPW = 8
_ROWS = _NW * _CPW


def _sc_gather(p0f, p1f, p2f, idx_all):
    mesh = plsc.VectorSubcoreMesh(core_axis_name="c", subcore_axis_name="s")

    @functools.partial(
        pl.kernel,
        mesh=mesh,
        out_type=jax.ShapeDtypeStruct((3, _ROWS, _CHUNK), jnp.float32),
        scratch_types=[
            pltpu.VMEM((_CPW, _CHUNK), jnp.int32),
            pltpu.VMEM((_CPW, _CHUNK), jnp.int32),
            pltpu.VMEM((_CPW, _CHUNK), jnp.int32),
            pltpu.VMEM((_CPW, _CHUNK), jnp.float32),
            pltpu.VMEM((_CPW, _CHUNK), jnp.float32),
            pltpu.VMEM((_CPW, _CHUNK), jnp.float32),
            pltpu.SemaphoreType.DMA,
        ],
    )
    def k(p0, p1, p2, idx_hbm, out_hbm, i0, i1, i2, g0, g1, g2, sem):
        wid = lax.axis_index("s") * 2 + lax.axis_index("c")
        base = wid * _CPW
        preds = (p0, p1, p2)
        ivs = (i0, i1, i2)
        gvs = (g0, g1, g2)
        for s in range(3):
            pltpu.sync_copy(idx_hbm.at[s, pl.ds(base, _CPW)], ivs[s])
        copies = []
        for s in range(3):
            for j in range(_CPW):
                copies.append(
                    pltpu.async_copy(preds[s].at[ivs[s].at[j]], gvs[s].at[j], sem))
        for c in copies:
            c.wait()
        for s in range(3):
            pltpu.sync_copy(gvs[s], out_hbm.at[s, pl.ds(base, _CPW)])

    return k(p0f, p1f, p2f, idx_all)


def _f_bce(x):
    return jnp.maximum(x, 0.0) + jnp.log1p(jnp.exp(-jnp.abs(x)))


def _sigmoid(x):
    return 1.0 / (1.0 + jnp.exp(-x))


def _atan_pos(x):
    inv = x > 1.0
    t = jnp.where(inv, 1.0 / jnp.maximum(x, 1e-30), x)
    t2 = t * t
    p = -0.01172120
    p = p * t2 + 0.05265332
    p = p * t2 - 0.11643287
    p = p * t2 + 0.19354346
    p = p * t2 - 0.33262347
    p = p * t2 + 0.99997726
    r = t * p
    return jnp.where(inv, (math.pi / 2) - r, r)


def _ciou_cols(bx, by, bw, bh, tx, ty, tw, th, eps=1e-7):
    b1x1 = bx - bw / 2; b1x2 = bx + bw / 2
    b1y1 = by - bh / 2; b1y2 = by + bh / 2
    b2x1 = tx - tw / 2; b2x2 = tx + tw / 2
    b2y1 = ty - th / 2; b2y2 = ty + th / 2
    inter = (jnp.maximum(jnp.minimum(b1x2, b2x2) - jnp.maximum(b1x1, b2x1), 0.0)
             * jnp.maximum(jnp.minimum(b1y2, b2y2) - jnp.maximum(b1y1, b2y1), 0.0))
    w1 = b1x2 - b1x1; h1 = b1y2 - b1y1 + eps
    w2 = b2x2 - b2x1; h2 = b2y2 - b2y1 + eps
    union = w1 * h1 + w2 * h2 - inter + eps
    iou = inter / union
    cw = jnp.maximum(b1x2, b2x2) - jnp.minimum(b1x1, b2x1)
    ch = jnp.maximum(b1y2, b2y2) - jnp.minimum(b1y1, b2y1)
    c2 = cw ** 2 + ch ** 2 + eps
    rho2 = ((b2x1 + b2x2 - b1x1 - b1x2) ** 2
            + (b2y1 + b2y2 - b1y1 - b1y2) ** 2) / 4
    v = 4.0 / math.pi ** 2 * (_atan_pos(w2 / h2) - _atan_pos(w1 / h1)) ** 2
    alpha = v / (v - iou + (1.0 + eps))
    return iou - (rho2 / c2 + v * alpha)


def _scale_terms(pp, tb, an, gif, gjf, tcls, kc, kr, hw):
    px = pp[:, 0:1]; py = pp[:, 1:2]
    pw = pp[:, 2:3]; ph = pp[:, 3:4]; pobj = pp[:, 4:5]
    xy_x = _sigmoid(px) * 2.0 - 0.5
    xy_y = _sigmoid(py) * 2.0 - 0.5
    wh_w = (_sigmoid(pw) * 2.0) ** 2 * an[:, 0:1]
    wh_h = (_sigmoid(ph) * 2.0) ** 2 * an[:, 1:2]
    fs = float(hw)
    tx = tb[:, 0:1] * fs - gif
    ty = tb[:, 1:2] * fs - gjf
    tw = tb[:, 2:3] * fs
    th = tb[:, 3:4] * fs
    iou = _ciou_cols(xy_x, xy_y, wh_w, wh_h, tx, ty, tw, th)
    box_loss = 1.0 - jnp.sum(iou) / float(_N)
    keq = kc == kr
    ncol = lax.broadcasted_iota(jnp.int32, (_N, _N), 0)
    mrow = lax.broadcasted_iota(jnp.int32, (_N, _N), 1)
    later = jnp.where(keq & (mrow > ncol), 1.0, 0.0)
    has_later = jnp.sum(later, axis=1, keepdims=True)
    keep = jnp.where(has_later > 0.0, 1.0, 0.0)
    corr = jnp.sum(keep * pobj * jnp.maximum(iou, 0.0))
    cl = pp[:, 5:85]
    sumf = jnp.sum(_f_bce(cl))
    cm = lax.broadcasted_iota(jnp.int32, (_N, _NCLS), 1)
    pick = jnp.sum(jnp.where(cm == tcls, cl, 0.0))
    cls_loss = (sumf - pick) / float(_N * _NCLS)
    return box_loss, corr, cls_loss


def _tc_body(p0, p1, p2, pp0, pp1, pp2, tb0, tb1, tb2, an0, an1, an2,
             gi0, gj0, tc0, kc0, kr0,
             gi1, gj1, tc1, kc1, kr1,
             gi2, gj2, tc2, kc2, kr2, out_ref):
    a = pl.program_id(0)
    plane = jnp.zeros((1, 1), jnp.float32)
    for s, pref in enumerate((p0, p1, p2)):
        hw = _HW[s]
        norm = _BAL[s] / float(16 * 3 * hw * hw)
        plane = plane + norm * jnp.sum(pref[...]).reshape(1, 1)

    @pl.when(a == 0)
    def _():
        out_ref[...] = plane

    @pl.when(a != 0)
    def _():
        out_ref[...] = out_ref[...] + plane

    @pl.when(a == 2)
    def _():
        extra = jnp.zeros((1, 1), jnp.float32)
        per_scale = (
            (pp0, tb0, an0, gi0, gj0, tc0, kc0, kr0),
            (pp1, tb1, an1, gi1, gj1, tc1, kc1, kr1),
            (pp2, tb2, an2, gi2, gj2, tc2, kc2, kr2),
        )
        for s in range(3):
            pp, tb, an, gi, gj, tc, kc, kr = per_scale[s]
            hw = _HW[s]
            box_l, corr, cls_l = _scale_terms(
                pp[...], tb[...], an[...],
                gi[...].astype(jnp.float32), gj[...].astype(jnp.float32),
                tc[...], kc[...], kr[...], hw)
            norm = _BAL[s] / float(16 * 3 * hw * hw)
            extra = extra + (0.05 * box_l - norm * corr
                             + 0.5 * cls_l).reshape(1, 1)
        out_ref[...] = out_ref[...] + extra


def _tc_body_planesum(pref):
    return jnp.sum(_f_bce(pref))


def kernel(pred0, pred1, pred2, tbox0, tbox1, tbox2, anch0, anch1, anch2,
           b0, a0, gj0, gi0, tcls0, b1, a1, gj1, gi1, tcls1,
           b2, a2, gj2, gi2, tcls2):
    preds = (pred0, pred1, pred2)
    bs = (b0, b1, b2); anchsel = (a0, a1, a2)
    gjs = (gj0, gj1, gj2); gis = (gi0, gi1, gi2)

    idx_list, kcs, krs, gifs, gjfs = [], [], [], [], []
    for s, hw in enumerate(_HW):
        b = bs[s].astype(jnp.int32)
        a = anchsel[s].astype(jnp.int32)
        gj = gjs[s].astype(jnp.int32)
        gi = gis[s].astype(jnp.int32)
        base = ((b * 255 + a * 85) * hw + gj) * hw + gi
        idx = base[:, None] + (jnp.arange(_C, dtype=jnp.int32) * (hw * hw))[None, :]
        idx = jnp.concatenate(
            [idx.reshape(-1),
             jnp.zeros((_ROWS * _CHUNK - _N * _C,), jnp.int32)])
        idx_list.append(idx.reshape(_ROWS, _CHUNK))
        key = ((b * 3 + a) * hw + gj) * hw + gi
        kcs.append(key[:, None])
        krs.append(key[None, :])
        gifs.append(gi[:, None])
        gjfs.append(gj[:, None])
    idx_all = jnp.stack(idx_list)

    gath = _sc_gather(pred0.reshape(-1), pred1.reshape(-1), pred2.reshape(-1),
                      idx_all)
    pps = gath.reshape(3, _ROWS * _CHUNK)[:, :_N * _C].reshape(3, _N, _C)

    tcs = (tcls0[:, None].astype(jnp.int32),
           tcls1[:, None].astype(jnp.int32),
           tcls2[:, None].astype(jnp.int32))

    def fused(p0r, p1r, p2r, pp0r, pp1r, pp2r, tb0r, tb1r, tb2r,
              an0r, an1r, an2r,
              gi0r, gj0r, tc0r, kc0r, kr0r,
              gi1r, gj1r, tc1r, kc1r, kr1r,
              gi2r, gj2r, tc2r, kc2r, kr2r, out_ref):
        _tc_body(_FRef(p0r), _FRef(p1r), _FRef(p2r),
                 pp0r, pp1r, pp2r, tb0r, tb1r, tb2r, an0r, an1r, an2r,
                 gi0r, gj0r, tc0r, kc0r, kr0r,
                 gi1r, gj1r, tc1r, kc1r, kr1r,
                 gi2r, gj2r, tc2r, kc2r, kr2r, out_ref)

    full = lambda shape: pl.BlockSpec(shape, lambda a: tuple(0 for _ in shape))
    plane_spec = lambda hw: pl.BlockSpec((16, 1, hw, hw),
                                         lambda a: (0, 85 * a + 4, 0, 0))
    out = pl.pallas_call(
        fused,
        grid=(3,),
        in_specs=[
            plane_spec(20), plane_spec(40), plane_spec(80),
            full((_N, _C)), full((_N, _C)), full((_N, _C)),
            full((_N, 4)), full((_N, 4)), full((_N, 4)),
            full((_N, 2)), full((_N, 2)), full((_N, 2)),
            full((_N, 1)), full((_N, 1)), full((_N, 1)), full((_N, 1)), full((1, _N)),
            full((_N, 1)), full((_N, 1)), full((_N, 1)), full((_N, 1)), full((1, _N)),
            full((_N, 1)), full((_N, 1)), full((_N, 1)), full((_N, 1)), full((1, _N)),
        ],
        out_specs=pl.BlockSpec((1, 1), lambda a: (0, 0)),
        out_shape=jax.ShapeDtypeStruct((1, 1), jnp.float32),
    )(pred0, pred1, pred2, pps[0], pps[1], pps[2],
      tbox0, tbox1, tbox2, anch0, anch1, anch2,
      gifs[0], gjfs[0], tcs[0], kcs[0], krs[0],
      gifs[1], gjfs[1], tcs[1], kcs[1], krs[1],
      gifs[2], gjfs[2], tcs[2], kcs[2], krs[2])
    return out.reshape(1)


class _FRef:
    def __init__(self, ref):
        self._ref = ref

    def __getitem__(self, idx):
        return _f_bce(self._ref[idx])


# speedup vs baseline: 5.6875x; 3.7963x over previous
import functools
import math

import jax
import jax.numpy as jnp
from jax import lax
from jax.experimental import pallas as pl
from jax.experimental.pallas import tpu as pltpu
from jax.experimental.pallas import tpu_sc as plsc

_HW = (20, 40, 80)
_N = 300
_NCH = 255
_NCLS = 80
_BAL = (0.4, 1.0, 4.0)
_NW = 32
_RPW = 16
_RPAD = _NW * _RPW
_G1 = 25


def _sc_gather_lo(v0, v1, v2, ridx):
    mesh = plsc.VectorSubcoreMesh(core_axis_name="c", subcore_axis_name="s")

    @functools.partial(
        pl.kernel,
        mesh=mesh,
        out_type=jax.ShapeDtypeStruct((3, _RPAD, 128), jnp.float32),
        scratch_types=[
            pltpu.VMEM((3, _RPAD), jnp.int32),
            pltpu.VMEM((_RPW, 128), jnp.float32),
            pltpu.VMEM((_RPW, 128), jnp.float32),
            pltpu.VMEM((_RPW, 128), jnp.float32),
            pltpu.SemaphoreType.DMA,
        ],
    )
    def k(t0, t1, t2, ridx_hbm, out_hbm, iv, s0, s1, s2, sem):
        wid = lax.axis_index("s") * 2 + lax.axis_index("c")
        base = wid * _RPW
        pltpu.sync_copy(ridx_hbm, iv)
        copies = []
        for s, (tbl, buf) in enumerate(((t0, s0), (t1, s1), (t2, s2))):
            isl = iv.at[s, pl.ds(base, _RPW)]
            copies.append(
                pltpu.async_copy(tbl.at[isl, pl.ds(0, 128)], buf, sem))
        for c in copies:
            c.wait()
        for s, buf in enumerate((s0, s1, s2)):
            pltpu.sync_copy(buf, out_hbm.at[s, pl.ds(base, _RPW)])

    return k(v0, v1, v2, ridx)


def _f_bce(x):
    return jnp.maximum(x, 0.0) + jnp.log1p(jnp.exp(-jnp.abs(x)))


def _sigmoid(x):
    return 1.0 / (1.0 + jnp.exp(-x))


def _atan_pos(x):
    inv = x > 1.0
    t = jnp.where(inv, 1.0 / jnp.maximum(x, 1e-30), x)
    t2 = t * t
    p = -0.01172120
    p = p * t2 + 0.05265332
    p = p * t2 - 0.11643287
    p = p * t2 + 0.19354346
    p = p * t2 - 0.33262347
    p = p * t2 + 0.99997726
    r = t * p
    return jnp.where(inv, (math.pi / 2) - r, r)


def _ciou_cols(bx, by, bw, bh, tx, ty, tw, th, eps=1e-7):
    b1x1 = bx - bw / 2; b1x2 = bx + bw / 2
    b1y1 = by - bh / 2; b1y2 = by + bh / 2
    b2x1 = tx - tw / 2; b2x2 = tx + tw / 2
    b2y1 = ty - th / 2; b2y2 = ty + th / 2
    inter = (jnp.maximum(jnp.minimum(b1x2, b2x2) - jnp.maximum(b1x1, b2x1), 0.0)
             * jnp.maximum(jnp.minimum(b1y2, b2y2) - jnp.maximum(b1y1, b2y1), 0.0))
    w1 = b1x2 - b1x1; h1 = b1y2 - b1y1 + eps
    w2 = b2x2 - b2x1; h2 = b2y2 - b2y1 + eps
    union = w1 * h1 + w2 * h2 - inter + eps
    iou = inter / union
    cw = jnp.maximum(b1x2, b2x2) - jnp.minimum(b1x1, b2x1)
    ch = jnp.maximum(b1y2, b2y2) - jnp.minimum(b1y1, b2y1)
    c2 = cw ** 2 + ch ** 2 + eps
    rho2 = ((b2x1 + b2x2 - b1x1 - b1x2) ** 2
            + (b2y1 + b2y2 - b1y1 - b1y2) ** 2) / 4
    v = 4.0 / math.pi ** 2 * (_atan_pos(w2 / h2) - _atan_pos(w1 / h1)) ** 2
    alpha = v / (v - iou + (1.0 + eps))
    return iou - (rho2 / c2 + v * alpha)


def _scale_terms(pp, tb, an, gif, gjf, tcls, kc, kr, hw):
    px = pp[:, 0:1]; py = pp[:, 1:2]
    pw = pp[:, 2:3]; ph = pp[:, 3:4]; pobj = pp[:, 4:5]
    xy_x = _sigmoid(px) * 2.0 - 0.5
    xy_y = _sigmoid(py) * 2.0 - 0.5
    wh_w = (_sigmoid(pw) * 2.0) ** 2 * an[:, 0:1]
    wh_h = (_sigmoid(ph) * 2.0) ** 2 * an[:, 1:2]
    fs = float(hw)
    tx = tb[:, 0:1] * fs - gif
    ty = tb[:, 1:2] * fs - gjf
    tw = tb[:, 2:3] * fs
    th = tb[:, 3:4] * fs
    iou = _ciou_cols(xy_x, xy_y, wh_w, wh_h, tx, ty, tw, th)
    box_loss = 1.0 - jnp.sum(iou) / float(_N)
    keq = kc == kr
    ncol = lax.broadcasted_iota(jnp.int32, (_N, _N), 0)
    mrow = lax.broadcasted_iota(jnp.int32, (_N, _N), 1)
    later = jnp.where(keq & (mrow > ncol), 1.0, 0.0)
    has_later = jnp.sum(later, axis=1, keepdims=True)
    keep = jnp.where(has_later > 0.0, 1.0, 0.0)
    corr = jnp.sum(keep * pobj * jnp.maximum(iou, 0.0))
    cl = pp[:, 5:85]
    sumf = jnp.sum(_f_bce(cl))
    cm = lax.broadcasted_iota(jnp.int32, (_N, _NCLS), 1)
    pick = jnp.sum(jnp.where(cm == tcls, cl, 0.0))
    cls_loss = (sumf - pick) / float(_N * _NCLS)
    return box_loss, corr, cls_loss


def _tc1_body(x0, x1, x2, out_ref):
    i = pl.program_id(0)
    sel = jnp.where(
        lax.broadcasted_iota(jnp.int32, (_NCH, 3), 0)
        == 4 + 85 * lax.broadcasted_iota(jnp.int32, (_NCH, 3), 1), 1.0, 0.0)
    acc = jnp.zeros((1, 1), jnp.float32)
    for s, x in enumerate((x0, x1, x2)):
        hw = _HW[s]
        norm = _BAL[s] / float(16 * 3 * hw * hw)
        cols = lax.dot_general(sel, x[...], (((0,), (1,)), ((), ())),
                               preferred_element_type=jnp.float32)
        acc = acc + norm * jnp.sum(_f_bce(cols)).reshape(1, 1)

    @pl.when(i == 0)
    def _():
        out_ref[...] = acc

    @pl.when(i != 0)
    def _():
        out_ref[...] = out_ref[...] + acc


_RING = 4


def _tc2_body(bands_sm, rmods_sm, va0, va1, va2, lo0, lo1, lo2,
              tb0, tb1, tb2, an0, an1, an2,
              a0, gi0, gj0, tc0, kc0, kr0,
              a1, gi1, gj1, tc1, kc1, kr1,
              a2, gi2, gj2, tc2, kc2, kr2, plane, out_ref,
              ring0, ring1, ring2, hi0, hi1, hi2, sems):
    vs = (va0, va1, va2)
    rings = (ring0, ring1, ring2)
    his = (hi0, hi1, hi2)

    def start(s, i):
        band = bands_sm[s, i]
        slot = lax.rem(i, _RING)
        return pltpu.make_async_copy(
            vs[s].at[pl.ds(pl.multiple_of(band * 8, 8), 8), pl.ds(128, 127)],
            rings[s].at[slot], sems.at[s, slot])

    def body(i, carry):
        @pl.when(i >= _RING)
        def _():
            j = i - _RING
            slot = lax.rem(j, _RING)
            for s in range(3):
                start(s, j).wait()
                rm = rmods_sm[s, j]
                row = rings[s][pl.ds(slot, 1), pl.ds(rm, 1), :]
                his[s][pl.ds(j, 1), :] = row.reshape(1, 127)

        @pl.when(i < _N)
        def _():
            for s in range(3):
                start(s, i).start()
        return carry

    lax.fori_loop(0, _N + _RING, body, 0)

    total = plane[...]
    per_scale = (
        (lo0, hi0, tb0, an0, a0, gi0, gj0, tc0, kc0, kr0),
        (lo1, hi1, tb1, an1, a1, gi1, gj1, tc1, kc1, kr1),
        (lo2, hi2, tb2, an2, a2, gi2, gj2, tc2, kc2, kr2),
    )
    for s in range(3):
        lo, hi, tb, an, a, gi, gj, tc, kc, kr = per_scale[s]
        hw = _HW[s]
        rows = jnp.concatenate([lo[...][0:_N], hi[...][0:_N]], axis=1)
        av = a[...]
        pp = jnp.where(av == 0, rows[:, 0:85],
                       jnp.where(av == 1, rows[:, 85:170], rows[:, 170:255]))
        box_l, corr, cls_l = _scale_terms(
            pp, tb[...], an[...],
            gi[...].astype(jnp.float32), gj[...].astype(jnp.float32),
            tc[...], kc[...], kr[...], hw)
        norm = _BAL[s] / float(16 * 3 * hw * hw)
        total = total + (0.05 * box_l - norm * corr
                         + 0.5 * cls_l).reshape(1, 1)
    out_ref[...] = total


def kernel(pred0, pred1, pred2, tbox0, tbox1, tbox2, anch0, anch1, anch2,
           b0, a0, gj0, gi0, tcls0, b1, a1, gj1, gi1, tcls1,
           b2, a2, gj2, gi2, tcls2):
    v0 = pred0.transpose(2, 3, 0, 1).reshape(20 * 20 * 16, _NCH)
    v1 = pred1.transpose(0, 2, 3, 1).reshape(16 * 40 * 40, _NCH)
    v2 = pred2.transpose(0, 2, 3, 1).reshape(16 * 80 * 80, _NCH)

    bs = (b0, b1, b2); ans = (a0, a1, a2)
    gjs = (gj0, gj1, gj2); gis = (gi0, gi1, gi2)
    tcls = (tcls0, tcls1, tcls2)

    ridx, bidx, rmods, kcs, krs, acols, gifs, gjfs, tccols = (
        [], [], [], [], [], [], [], [], [])
    for s, hw in enumerate(_HW):
        b = bs[s].astype(jnp.int32)
        a = ans[s].astype(jnp.int32)
        gj = gjs[s].astype(jnp.int32)
        gi = gis[s].astype(jnp.int32)
        if s == 0:
            r = (gj * hw + gi) * 16 + b
        else:
            r = (b * hw + gj) * hw + gi
        pad = jnp.zeros((_RPAD - _N,), jnp.int32)
        ridx.append(jnp.concatenate([r, pad]))
        bidx.append(jnp.concatenate([r // 8, pad]))
        rmods.append(jnp.concatenate([r % 8, pad]))
        key = ((b * 3 + a) * hw + gj) * hw + gi
        kcs.append(key[:, None])
        krs.append(key[None, :])
        acols.append(a[:, None])
        gifs.append(gi[:, None])
        gjfs.append(gj[:, None])
        tccols.append(tcls[s][:, None].astype(jnp.int32))
    ridx_all = jnp.stack(ridx)
    bidx_all = jnp.stack(bidx)
    rmod_all = jnp.stack(rmods)

    lo = _sc_gather_lo(v0, v1, v2, ridx_all)

    blk = lambda n: pl.BlockSpec((n, _NCH), lambda i: (i, 0))
    plane = pl.pallas_call(
        _tc1_body,
        grid=(_G1,),
        in_specs=[blk(6400 // _G1), blk(25600 // _G1), blk(102400 // _G1)],
        out_specs=pl.BlockSpec((1, 1), lambda i: (0, 0)),
        out_shape=jax.ShapeDtypeStruct((1, 1), jnp.float32),
    )(v0, v1, v2)

    full = lambda shape: pl.BlockSpec(shape, lambda: tuple(0 for _ in shape))
    out = pl.pallas_call(
        _tc2_body,
        in_specs=[
            pl.BlockSpec(memory_space=pltpu.MemorySpace.SMEM),
            pl.BlockSpec(memory_space=pltpu.MemorySpace.SMEM),
            pl.BlockSpec(memory_space=pl.ANY),
            pl.BlockSpec(memory_space=pl.ANY),
            pl.BlockSpec(memory_space=pl.ANY),
            full((_RPAD, 128)), full((_RPAD, 128)), full((_RPAD, 128)),
            full((_N, 4)), full((_N, 4)), full((_N, 4)),
            full((_N, 2)), full((_N, 2)), full((_N, 2)),
            full((_N, 1)), full((_N, 1)), full((_N, 1)), full((_N, 1)),
            full((_N, 1)), full((1, _N)),
            full((_N, 1)), full((_N, 1)), full((_N, 1)), full((_N, 1)),
            full((_N, 1)), full((1, _N)),
            full((_N, 1)), full((_N, 1)), full((_N, 1)), full((_N, 1)),
            full((_N, 1)), full((1, _N)),
            full((1, 1)),
        ],
        out_specs=full((1, 1)),
        out_shape=jax.ShapeDtypeStruct((1, 1), jnp.float32),
        scratch_shapes=[
            pltpu.VMEM((_RING, 8, 127), jnp.float32),
            pltpu.VMEM((_RING, 8, 127), jnp.float32),
            pltpu.VMEM((_RING, 8, 127), jnp.float32),
            pltpu.VMEM((_N, 127), jnp.float32),
            pltpu.VMEM((_N, 127), jnp.float32),
            pltpu.VMEM((_N, 127), jnp.float32),
            pltpu.SemaphoreType.DMA((3, _RING)),
        ],
    )(bidx_all, rmod_all, v0, v1, v2, lo[0], lo[1], lo[2],
      tbox0, tbox1, tbox2, anch0, anch1, anch2,
      acols[0], gifs[0], gjfs[0], tccols[0], kcs[0], krs[0],
      acols[1], gifs[1], gjfs[1], tccols[1], kcs[1], krs[1],
      acols[2], gifs[2], gjfs[2], tccols[2], kcs[2], krs[2],
      plane)
    return out.reshape(1)


# speedup vs baseline: 6.7688x; 1.1901x over previous
import functools
import math

import jax
import jax.numpy as jnp
from jax import lax
from jax.experimental import pallas as pl
from jax.experimental.pallas import tpu as pltpu
from jax.experimental.pallas import tpu_sc as plsc

_HW = (20, 40, 80)
_N = 300
_NCH = 255
_NCLS = 80
_BAL = (0.4, 1.0, 4.0)
_NW = 32
_RPW = 16
_RPAD = _NW * _RPW
_G1 = 25


def _sc_gather_lo(v0, v1, v2, ridx):
    mesh = plsc.VectorSubcoreMesh(core_axis_name="c", subcore_axis_name="s")

    @functools.partial(
        pl.kernel,
        mesh=mesh,
        out_type=jax.ShapeDtypeStruct((3, _RPAD, 128), jnp.float32),
        scratch_types=[
            pltpu.VMEM((3, _RPAD), jnp.int32),
            pltpu.VMEM((_RPW, 128), jnp.float32),
            pltpu.VMEM((_RPW, 128), jnp.float32),
            pltpu.VMEM((_RPW, 128), jnp.float32),
            pltpu.SemaphoreType.DMA,
        ],
    )
    def k(t0, t1, t2, ridx_hbm, out_hbm, iv, s0, s1, s2, sem):
        wid = lax.axis_index("s") * 2 + lax.axis_index("c")
        base = wid * _RPW
        pltpu.sync_copy(ridx_hbm, iv)
        copies = []
        for s, (tbl, buf) in enumerate(((t0, s0), (t1, s1), (t2, s2))):
            isl = iv.at[s, pl.ds(base, _RPW)]
            copies.append(
                pltpu.async_copy(tbl.at[isl, pl.ds(0, 128)], buf, sem))
        for c in copies:
            c.wait()
        for s, buf in enumerate((s0, s1, s2)):
            pltpu.sync_copy(buf, out_hbm.at[s, pl.ds(base, _RPW)])

    return k(v0, v1, v2, ridx)


def _f_bce(x):
    return jnp.maximum(x, 0.0) + jnp.log1p(jnp.exp(-jnp.abs(x)))


def _sigmoid(x):
    return 1.0 / (1.0 + jnp.exp(-x))


def _atan_pos(x):
    inv = x > 1.0
    t = jnp.where(inv, 1.0 / jnp.maximum(x, 1e-30), x)
    t2 = t * t
    p = -0.01172120
    p = p * t2 + 0.05265332
    p = p * t2 - 0.11643287
    p = p * t2 + 0.19354346
    p = p * t2 - 0.33262347
    p = p * t2 + 0.99997726
    r = t * p
    return jnp.where(inv, (math.pi / 2) - r, r)


def _ciou_cols(bx, by, bw, bh, tx, ty, tw, th, eps=1e-7):
    b1x1 = bx - bw / 2; b1x2 = bx + bw / 2
    b1y1 = by - bh / 2; b1y2 = by + bh / 2
    b2x1 = tx - tw / 2; b2x2 = tx + tw / 2
    b2y1 = ty - th / 2; b2y2 = ty + th / 2
    inter = (jnp.maximum(jnp.minimum(b1x2, b2x2) - jnp.maximum(b1x1, b2x1), 0.0)
             * jnp.maximum(jnp.minimum(b1y2, b2y2) - jnp.maximum(b1y1, b2y1), 0.0))
    w1 = b1x2 - b1x1; h1 = b1y2 - b1y1 + eps
    w2 = b2x2 - b2x1; h2 = b2y2 - b2y1 + eps
    union = w1 * h1 + w2 * h2 - inter + eps
    iou = inter / union
    cw = jnp.maximum(b1x2, b2x2) - jnp.minimum(b1x1, b2x1)
    ch = jnp.maximum(b1y2, b2y2) - jnp.minimum(b1y1, b2y1)
    c2 = cw ** 2 + ch ** 2 + eps
    rho2 = ((b2x1 + b2x2 - b1x1 - b1x2) ** 2
            + (b2y1 + b2y2 - b1y1 - b1y2) ** 2) / 4
    v = 4.0 / math.pi ** 2 * (_atan_pos(w2 / h2) - _atan_pos(w1 / h1)) ** 2
    alpha = v / (v - iou + (1.0 + eps))
    return iou - (rho2 / c2 + v * alpha)


def _scale_terms(pp, tb, an, gif, gjf, tcls, kc, kr, hw):
    px = pp[:, 0:1]; py = pp[:, 1:2]
    pw = pp[:, 2:3]; ph = pp[:, 3:4]; pobj = pp[:, 4:5]
    xy_x = _sigmoid(px) * 2.0 - 0.5
    xy_y = _sigmoid(py) * 2.0 - 0.5
    wh_w = (_sigmoid(pw) * 2.0) ** 2 * an[:, 0:1]
    wh_h = (_sigmoid(ph) * 2.0) ** 2 * an[:, 1:2]
    fs = float(hw)
    tx = tb[:, 0:1] * fs - gif
    ty = tb[:, 1:2] * fs - gjf
    tw = tb[:, 2:3] * fs
    th = tb[:, 3:4] * fs
    iou = _ciou_cols(xy_x, xy_y, wh_w, wh_h, tx, ty, tw, th)
    box_loss = 1.0 - jnp.sum(iou) / float(_N)
    keq = kc == kr
    ncol = lax.broadcasted_iota(jnp.int32, (_N, _N), 0)
    mrow = lax.broadcasted_iota(jnp.int32, (_N, _N), 1)
    later = jnp.where(keq & (mrow > ncol), 1.0, 0.0)
    has_later = jnp.sum(later, axis=1, keepdims=True)
    keep = jnp.where(has_later > 0.0, 1.0, 0.0)
    corr = jnp.sum(keep * pobj * jnp.maximum(iou, 0.0))
    cl = pp[:, 5:85]
    sumf = jnp.sum(_f_bce(cl))
    cm = lax.broadcasted_iota(jnp.int32, (_N, _NCLS), 1)
    pick = jnp.sum(jnp.where(cm == tcls, cl, 0.0))
    cls_loss = (sumf - pick) / float(_N * _NCLS)
    return box_loss, corr, cls_loss


def _tc1_body(x0, x1, x2, out_ref):
    i = pl.program_id(0)
    sel_lo = jnp.where(
        lax.broadcasted_iota(jnp.int32, (128, 2), 0)
        == 4 + 85 * lax.broadcasted_iota(jnp.int32, (128, 2), 1), 1.0, 0.0)
    sel_hi = jnp.where(
        lax.broadcasted_iota(jnp.int32, (127, 1), 0) == 46, 1.0, 0.0)
    acc = jnp.zeros((1, 1), jnp.float32)
    dn = (((0,), (1,)), ((), ()))
    for s, x in enumerate((x0, x1, x2)):
        hw = _HW[s]
        norm = _BAL[s] / float(16 * 3 * hw * hw)
        cols_lo = lax.dot_general(sel_lo, x[:, 0:128], dn,
                                  preferred_element_type=jnp.float32)
        cols_hi = lax.dot_general(sel_hi, x[:, 128:255], dn,
                                  preferred_element_type=jnp.float32)
        acc = acc + norm * (jnp.sum(_f_bce(cols_lo))
                            + jnp.sum(_f_bce(cols_hi))).reshape(1, 1)

    @pl.when(i == 0)
    def _():
        out_ref[...] = acc

    @pl.when(i != 0)
    def _():
        out_ref[...] = out_ref[...] + acc


_RING = 8


def _tc2_body(bands_sm, rmods_sm, va0, va1, va2, lo0, lo1, lo2,
              tb0, tb1, tb2, an0, an1, an2,
              a0, gi0, gj0, tc0, kc0, kr0,
              a1, gi1, gj1, tc1, kc1, kr1,
              a2, gi2, gj2, tc2, kc2, kr2, plane, out_ref,
              ring0, ring1, ring2, hi0, hi1, hi2, sems):
    vs = (va0, va1, va2)
    rings = (ring0, ring1, ring2)
    his = (hi0, hi1, hi2)

    def start(s, i):
        band = bands_sm[s, i]
        slot = lax.rem(i, _RING)
        return pltpu.make_async_copy(
            vs[s].at[pl.ds(pl.multiple_of(band * 8, 8), 8), pl.ds(128, 127)],
            rings[s].at[slot], sems.at[s, slot])

    def body(i, carry):
        for t in range(2):
            j = 2 * i + t - _RING

            @pl.when((j >= 0) & (j < _N))
            def _():
                slot = lax.rem(j, _RING)
                for s in range(3):
                    start(s, j).wait()
                    rm = rmods_sm[s, j]
                    row = rings[s][pl.ds(slot, 1), pl.ds(rm, 1), :]
                    his[s][pl.ds(j, 1), :] = row.reshape(1, 127)

        for t in range(2):
            k = 2 * i + t

            @pl.when(k < _N)
            def _():
                for s in range(3):
                    start(s, k).start()
        return carry

    lax.fori_loop(0, (_N + _RING) // 2, body, 0)

    total = plane[...]
    per_scale = (
        (lo0, hi0, tb0, an0, a0, gi0, gj0, tc0, kc0, kr0),
        (lo1, hi1, tb1, an1, a1, gi1, gj1, tc1, kc1, kr1),
        (lo2, hi2, tb2, an2, a2, gi2, gj2, tc2, kc2, kr2),
    )
    for s in range(3):
        lo, hi, tb, an, a, gi, gj, tc, kc, kr = per_scale[s]
        hw = _HW[s]
        rows = jnp.concatenate([lo[...][0:_N], hi[...][0:_N]], axis=1)
        av = a[...]
        pp = jnp.where(av == 0, rows[:, 0:85],
                       jnp.where(av == 1, rows[:, 85:170], rows[:, 170:255]))
        box_l, corr, cls_l = _scale_terms(
            pp, tb[...], an[...],
            gi[...].astype(jnp.float32), gj[...].astype(jnp.float32),
            tc[...], kc[...], kr[...], hw)
        norm = _BAL[s] / float(16 * 3 * hw * hw)
        total = total + (0.05 * box_l - norm * corr
                         + 0.5 * cls_l).reshape(1, 1)
    out_ref[...] = total


def kernel(pred0, pred1, pred2, tbox0, tbox1, tbox2, anch0, anch1, anch2,
           b0, a0, gj0, gi0, tcls0, b1, a1, gj1, gi1, tcls1,
           b2, a2, gj2, gi2, tcls2):
    v0 = pred0.transpose(2, 3, 0, 1).reshape(20 * 20 * 16, _NCH)
    v1 = pred1.transpose(0, 2, 3, 1).reshape(16 * 40 * 40, _NCH)
    v2 = pred2.transpose(0, 2, 3, 1).reshape(16 * 80 * 80, _NCH)

    bs = (b0, b1, b2); ans = (a0, a1, a2)
    gjs = (gj0, gj1, gj2); gis = (gi0, gi1, gi2)
    tcls = (tcls0, tcls1, tcls2)

    ridx, bidx, rmods, kcs, krs, acols, gifs, gjfs, tccols = (
        [], [], [], [], [], [], [], [], [])
    for s, hw in enumerate(_HW):
        b = bs[s].astype(jnp.int32)
        a = ans[s].astype(jnp.int32)
        gj = gjs[s].astype(jnp.int32)
        gi = gis[s].astype(jnp.int32)
        if s == 0:
            r = (gj * hw + gi) * 16 + b
        else:
            r = (b * hw + gj) * hw + gi
        pad = jnp.zeros((_RPAD - _N,), jnp.int32)
        ridx.append(jnp.concatenate([r, pad]))
        bidx.append(jnp.concatenate([r // 8, pad]))
        rmods.append(jnp.concatenate([r % 8, pad]))
        key = ((b * 3 + a) * hw + gj) * hw + gi
        kcs.append(key[:, None])
        krs.append(key[None, :])
        acols.append(a[:, None])
        gifs.append(gi[:, None])
        gjfs.append(gj[:, None])
        tccols.append(tcls[s][:, None].astype(jnp.int32))
    ridx_all = jnp.stack(ridx)
    bidx_all = jnp.stack(bidx)
    rmod_all = jnp.stack(rmods)

    lo = _sc_gather_lo(v0, v1, v2, ridx_all)

    blk = lambda n: pl.BlockSpec((n, _NCH), lambda i: (i, 0))
    plane = pl.pallas_call(
        _tc1_body,
        grid=(_G1,),
        in_specs=[blk(6400 // _G1), blk(25600 // _G1), blk(102400 // _G1)],
        out_specs=pl.BlockSpec((1, 1), lambda i: (0, 0)),
        out_shape=jax.ShapeDtypeStruct((1, 1), jnp.float32),
    )(v0, v1, v2)

    full = lambda shape: pl.BlockSpec(shape, lambda: tuple(0 for _ in shape))
    out = pl.pallas_call(
        _tc2_body,
        in_specs=[
            pl.BlockSpec(memory_space=pltpu.MemorySpace.SMEM),
            pl.BlockSpec(memory_space=pltpu.MemorySpace.SMEM),
            pl.BlockSpec(memory_space=pl.ANY),
            pl.BlockSpec(memory_space=pl.ANY),
            pl.BlockSpec(memory_space=pl.ANY),
            full((_RPAD, 128)), full((_RPAD, 128)), full((_RPAD, 128)),
            full((_N, 4)), full((_N, 4)), full((_N, 4)),
            full((_N, 2)), full((_N, 2)), full((_N, 2)),
            full((_N, 1)), full((_N, 1)), full((_N, 1)), full((_N, 1)),
            full((_N, 1)), full((1, _N)),
            full((_N, 1)), full((_N, 1)), full((_N, 1)), full((_N, 1)),
            full((_N, 1)), full((1, _N)),
            full((_N, 1)), full((_N, 1)), full((_N, 1)), full((_N, 1)),
            full((_N, 1)), full((1, _N)),
            full((1, 1)),
        ],
        out_specs=full((1, 1)),
        out_shape=jax.ShapeDtypeStruct((1, 1), jnp.float32),
        scratch_shapes=[
            pltpu.VMEM((_RING, 8, 127), jnp.float32),
            pltpu.VMEM((_RING, 8, 127), jnp.float32),
            pltpu.VMEM((_RING, 8, 127), jnp.float32),
            pltpu.VMEM((_N, 127), jnp.float32),
            pltpu.VMEM((_N, 127), jnp.float32),
            pltpu.VMEM((_N, 127), jnp.float32),
            pltpu.SemaphoreType.DMA((3, _RING)),
        ],
    )(bidx_all, rmod_all, v0, v1, v2, lo[0], lo[1], lo[2],
      tbox0, tbox1, tbox2, anch0, anch1, anch2,
      acols[0], gifs[0], gjfs[0], tccols[0], kcs[0], krs[0],
      acols[1], gifs[1], gjfs[1], tccols[1], kcs[1], krs[1],
      acols[2], gifs[2], gjfs[2], tccols[2], kcs[2], krs[2],
      plane)
    return out.reshape(1)


# speedup vs baseline: 7.6460x; 1.1296x over previous
import functools
import math

import jax
import jax.numpy as jnp
from jax import lax
from jax.experimental import pallas as pl
from jax.experimental.pallas import tpu as pltpu
from jax.experimental.pallas import tpu_sc as plsc

_HW = (20, 40, 80)
_N = 300
_NCH = 255
_NCLS = 80
_BAL = (0.4, 1.0, 4.0)
_NW = 32
_RPW = 16
_RPAD = _NW * _RPW
_G1 = 10


def _sc_gather_lo(v0, v1, v2, ridx):
    mesh = plsc.VectorSubcoreMesh(core_axis_name="c", subcore_axis_name="s")

    @functools.partial(
        pl.kernel,
        mesh=mesh,
        out_type=jax.ShapeDtypeStruct((3, _RPAD, 128), jnp.float32),
        scratch_types=[
            pltpu.VMEM((3, _RPAD), jnp.int32),
            pltpu.VMEM((_RPW, 128), jnp.float32),
            pltpu.VMEM((_RPW, 128), jnp.float32),
            pltpu.VMEM((_RPW, 128), jnp.float32),
            pltpu.SemaphoreType.DMA,
        ],
    )
    def k(t0, t1, t2, ridx_hbm, out_hbm, iv, s0, s1, s2, sem):
        wid = lax.axis_index("s") * 2 + lax.axis_index("c")
        base = wid * _RPW
        pltpu.sync_copy(ridx_hbm, iv)
        copies = []
        for s, (tbl, buf) in enumerate(((t0, s0), (t1, s1), (t2, s2))):
            isl = iv.at[s, pl.ds(base, _RPW)]
            copies.append(
                pltpu.async_copy(tbl.at[isl, pl.ds(0, 128)], buf, sem))
        for c in copies:
            c.wait()
        for s, buf in enumerate((s0, s1, s2)):
            pltpu.sync_copy(buf, out_hbm.at[s, pl.ds(base, _RPW)])

    return k(v0, v1, v2, ridx)


def _f_bce(x):
    return jnp.maximum(x, 0.0) + jnp.log1p(jnp.exp(-jnp.abs(x)))


def _sigmoid(x):
    return 1.0 / (1.0 + jnp.exp(-x))


def _atan_pos(x):
    inv = x > 1.0
    t = jnp.where(inv, 1.0 / jnp.maximum(x, 1e-30), x)
    t2 = t * t
    p = -0.01172120
    p = p * t2 + 0.05265332
    p = p * t2 - 0.11643287
    p = p * t2 + 0.19354346
    p = p * t2 - 0.33262347
    p = p * t2 + 0.99997726
    r = t * p
    return jnp.where(inv, (math.pi / 2) - r, r)


def _ciou_cols(bx, by, bw, bh, tx, ty, tw, th, eps=1e-7):
    b1x1 = bx - bw / 2; b1x2 = bx + bw / 2
    b1y1 = by - bh / 2; b1y2 = by + bh / 2
    b2x1 = tx - tw / 2; b2x2 = tx + tw / 2
    b2y1 = ty - th / 2; b2y2 = ty + th / 2
    inter = (jnp.maximum(jnp.minimum(b1x2, b2x2) - jnp.maximum(b1x1, b2x1), 0.0)
             * jnp.maximum(jnp.minimum(b1y2, b2y2) - jnp.maximum(b1y1, b2y1), 0.0))
    w1 = b1x2 - b1x1; h1 = b1y2 - b1y1 + eps
    w2 = b2x2 - b2x1; h2 = b2y2 - b2y1 + eps
    union = w1 * h1 + w2 * h2 - inter + eps
    iou = inter / union
    cw = jnp.maximum(b1x2, b2x2) - jnp.minimum(b1x1, b2x1)
    ch = jnp.maximum(b1y2, b2y2) - jnp.minimum(b1y1, b2y1)
    c2 = cw ** 2 + ch ** 2 + eps
    rho2 = ((b2x1 + b2x2 - b1x1 - b1x2) ** 2
            + (b2y1 + b2y2 - b1y1 - b1y2) ** 2) / 4
    v = 4.0 / math.pi ** 2 * (_atan_pos(w2 / h2) - _atan_pos(w1 / h1)) ** 2
    alpha = v / (v - iou + (1.0 + eps))
    return iou - (rho2 / c2 + v * alpha)


def _scale_terms(pp, tb, an, gif, gjf, tcls, kc, kr, hw):
    px = pp[:, 0:1]; py = pp[:, 1:2]
    pw = pp[:, 2:3]; ph = pp[:, 3:4]; pobj = pp[:, 4:5]
    xy_x = _sigmoid(px) * 2.0 - 0.5
    xy_y = _sigmoid(py) * 2.0 - 0.5
    wh_w = (_sigmoid(pw) * 2.0) ** 2 * an[:, 0:1]
    wh_h = (_sigmoid(ph) * 2.0) ** 2 * an[:, 1:2]
    fs = float(hw)
    tx = tb[:, 0:1] * fs - gif
    ty = tb[:, 1:2] * fs - gjf
    tw = tb[:, 2:3] * fs
    th = tb[:, 3:4] * fs
    iou = _ciou_cols(xy_x, xy_y, wh_w, wh_h, tx, ty, tw, th)
    box_loss = 1.0 - jnp.sum(iou) / float(_N)
    keq = kc == kr
    ncol = lax.broadcasted_iota(jnp.int32, (_N, _N), 0)
    mrow = lax.broadcasted_iota(jnp.int32, (_N, _N), 1)
    later = jnp.where(keq & (mrow > ncol), 1.0, 0.0)
    has_later = jnp.sum(later, axis=1, keepdims=True)
    keep = jnp.where(has_later > 0.0, 1.0, 0.0)
    corr = jnp.sum(keep * pobj * jnp.maximum(iou, 0.0))
    cl = pp[:, 5:85]
    sumf = jnp.sum(_f_bce(cl))
    cm = lax.broadcasted_iota(jnp.int32, (_N, _NCLS), 1)
    pick = jnp.sum(jnp.where(cm == tcls, cl, 0.0))
    cls_loss = (sumf - pick) / float(_N * _NCLS)
    return box_loss, corr, cls_loss


def _tc1_body(x0, x1, x2, out_ref):
    i = pl.program_id(0)
    sel_lo = jnp.where(
        lax.broadcasted_iota(jnp.int32, (128, 2), 0)
        == 4 + 85 * lax.broadcasted_iota(jnp.int32, (128, 2), 1), 1.0, 0.0)
    sel_hi = jnp.where(
        lax.broadcasted_iota(jnp.int32, (127, 1), 0) == 46, 1.0, 0.0)
    acc = jnp.zeros((1, 1), jnp.float32)
    dn = (((0,), (1,)), ((), ()))
    for s, x in enumerate((x0, x1, x2)):
        hw = _HW[s]
        norm = _BAL[s] / float(16 * 3 * hw * hw)
        cols_lo = lax.dot_general(sel_lo, x[:, 0:128], dn,
                                  preferred_element_type=jnp.float32)
        cols_hi = lax.dot_general(sel_hi, x[:, 128:255], dn,
                                  preferred_element_type=jnp.float32)
        acc = acc + norm * (jnp.sum(_f_bce(cols_lo))
                            + jnp.sum(_f_bce(cols_hi))).reshape(1, 1)

    @pl.when(i == 0)
    def _():
        out_ref[...] = acc

    @pl.when(i != 0)
    def _():
        out_ref[...] = out_ref[...] + acc


_RING = 16


def _tc2_body(bands_sm, va0, va1, va2, lo0, lo1, lo2,
              tb0, tb1, tb2, an0, an1, an2,
              a0, rm0, gi0, gj0, tc0, kc0, kr0,
              a1, rm1, gi1, gj1, tc1, kc1, kr1,
              a2, rm2, gi2, gj2, tc2, kc2, kr2, plane, out_ref,
              hi0, hi1, hi2, sems):
    vs = (va0, va1, va2)
    his = (hi0, hi1, hi2)

    def start(s, i):
        band = bands_sm[s, i]
        return pltpu.make_async_copy(
            vs[s].at[pl.ds(pl.multiple_of(band * 8, 8), 8), pl.ds(128, 127)],
            his[s].at[i], sems.at[s, lax.rem(i, _RING)])

    def body(i, carry):
        for t in range(2):
            j = 2 * i + t - _RING

            @pl.when((j >= 0) & (j < _N))
            def _():
                for s in range(3):
                    start(s, j).wait()

        for t in range(2):
            k = 2 * i + t

            @pl.when(k < _N)
            def _():
                for s in range(3):
                    start(s, k).start()
        return carry

    lax.fori_loop(0, (_N + _RING) // 2, body, 0)

    total = plane[...]
    per_scale = (
        (lo0, hi0, tb0, an0, a0, rm0, gi0, gj0, tc0, kc0, kr0),
        (lo1, hi1, tb1, an1, a1, rm1, gi1, gj1, tc1, kc1, kr1),
        (lo2, hi2, tb2, an2, a2, rm2, gi2, gj2, tc2, kc2, kr2),
    )
    for s in range(3):
        lo, hi, tb, an, a, rm, gi, gj, tc, kc, kr = per_scale[s]
        hw = _HW[s]
        rmask = jnp.where(
            lax.broadcasted_iota(jnp.int32, (_N, 8), 1) == rm[...],
            1.0, 0.0)
        hirows = jnp.sum(hi[...][0:_N] * rmask[:, :, None], axis=1)
        rows = jnp.concatenate([lo[...][0:_N], hirows], axis=1)
        av = a[...]
        pp = jnp.where(av == 0, rows[:, 0:85],
                       jnp.where(av == 1, rows[:, 85:170], rows[:, 170:255]))
        box_l, corr, cls_l = _scale_terms(
            pp, tb[...], an[...],
            gi[...].astype(jnp.float32), gj[...].astype(jnp.float32),
            tc[...], kc[...], kr[...], hw)
        norm = _BAL[s] / float(16 * 3 * hw * hw)
        total = total + (0.05 * box_l - norm * corr
                         + 0.5 * cls_l).reshape(1, 1)
    out_ref[...] = total


def kernel(pred0, pred1, pred2, tbox0, tbox1, tbox2, anch0, anch1, anch2,
           b0, a0, gj0, gi0, tcls0, b1, a1, gj1, gi1, tcls1,
           b2, a2, gj2, gi2, tcls2):
    v0 = pred0.transpose(2, 3, 0, 1).reshape(20 * 20 * 16, _NCH)
    v1 = pred1.transpose(0, 2, 3, 1).reshape(16 * 40 * 40, _NCH)
    v2 = pred2.transpose(0, 2, 3, 1).reshape(16 * 80 * 80, _NCH)

    bs = (b0, b1, b2); ans = (a0, a1, a2)
    gjs = (gj0, gj1, gj2); gis = (gi0, gi1, gi2)
    tcls = (tcls0, tcls1, tcls2)

    ridx, bidx, rmods, kcs, krs, acols, gifs, gjfs, tccols = (
        [], [], [], [], [], [], [], [], [])
    for s, hw in enumerate(_HW):
        b = bs[s].astype(jnp.int32)
        a = ans[s].astype(jnp.int32)
        gj = gjs[s].astype(jnp.int32)
        gi = gis[s].astype(jnp.int32)
        if s == 0:
            r = (gj * hw + gi) * 16 + b
        else:
            r = (b * hw + gj) * hw + gi
        pad = jnp.zeros((_RPAD - _N,), jnp.int32)
        ridx.append(jnp.concatenate([r, pad]))
        bidx.append(jnp.concatenate([r // 8, pad]))
        rmods.append((r % 8)[:, None])
        key = ((b * 3 + a) * hw + gj) * hw + gi
        kcs.append(key[:, None])
        krs.append(key[None, :])
        acols.append(a[:, None])
        gifs.append(gi[:, None])
        gjfs.append(gj[:, None])
        tccols.append(tcls[s][:, None].astype(jnp.int32))
    ridx_all = jnp.stack(ridx)
    bidx_all = jnp.stack(bidx)

    lo = _sc_gather_lo(v0, v1, v2, ridx_all)

    blk = lambda n: pl.BlockSpec((n, _NCH), lambda i: (i, 0))
    plane = pl.pallas_call(
        _tc1_body,
        grid=(_G1,),
        in_specs=[blk(6400 // _G1), blk(25600 // _G1), blk(102400 // _G1)],
        out_specs=pl.BlockSpec((1, 1), lambda i: (0, 0)),
        out_shape=jax.ShapeDtypeStruct((1, 1), jnp.float32),
    )(v0, v1, v2)

    full = lambda shape: pl.BlockSpec(shape, lambda: tuple(0 for _ in shape))
    out = pl.pallas_call(
        _tc2_body,
        in_specs=[
            pl.BlockSpec(memory_space=pltpu.MemorySpace.SMEM),
            pl.BlockSpec(memory_space=pl.ANY),
            pl.BlockSpec(memory_space=pl.ANY),
            pl.BlockSpec(memory_space=pl.ANY),
            full((_RPAD, 128)), full((_RPAD, 128)), full((_RPAD, 128)),
            full((_N, 4)), full((_N, 4)), full((_N, 4)),
            full((_N, 2)), full((_N, 2)), full((_N, 2)),
            full((_N, 1)), full((_N, 1)), full((_N, 1)), full((_N, 1)),
            full((_N, 1)), full((_N, 1)), full((1, _N)),
            full((_N, 1)), full((_N, 1)), full((_N, 1)), full((_N, 1)),
            full((_N, 1)), full((_N, 1)), full((1, _N)),
            full((_N, 1)), full((_N, 1)), full((_N, 1)), full((_N, 1)),
            full((_N, 1)), full((_N, 1)), full((1, _N)),
            full((1, 1)),
        ],
        out_specs=full((1, 1)),
        out_shape=jax.ShapeDtypeStruct((1, 1), jnp.float32),
        scratch_shapes=[
            pltpu.VMEM((_N, 8, 127), jnp.float32),
            pltpu.VMEM((_N, 8, 127), jnp.float32),
            pltpu.VMEM((_N, 8, 127), jnp.float32),
            pltpu.SemaphoreType.DMA((3, _RING)),
        ],
    )(bidx_all, v0, v1, v2, lo[0], lo[1], lo[2],
      tbox0, tbox1, tbox2, anch0, anch1, anch2,
      acols[0], rmods[0], gifs[0], gjfs[0], tccols[0], kcs[0], krs[0],
      acols[1], rmods[1], gifs[1], gjfs[1], tccols[1], kcs[1], krs[1],
      acols[2], rmods[2], gifs[2], gjfs[2], tccols[2], kcs[2], krs[2],
      plane)
    return out.reshape(1)


# speedup vs baseline: 7.6639x; 1.0023x over previous
import functools
import math

import jax
import jax.numpy as jnp
from jax import lax
from jax.experimental import pallas as pl
from jax.experimental.pallas import tpu as pltpu
from jax.experimental.pallas import tpu_sc as plsc

_HW = (20, 40, 80)
_N = 300
_NCH = 255
_NCLS = 80
_BAL = (0.4, 1.0, 4.0)
_NW = 32
_RPW = 16
_RPAD = _NW * _RPW
_G1 = 10


def _sc_gather_lo(v0, v1, v2, ridx):
    mesh = plsc.VectorSubcoreMesh(core_axis_name="c", subcore_axis_name="s")

    @functools.partial(
        pl.kernel,
        mesh=mesh,
        out_type=jax.ShapeDtypeStruct((3, _RPAD, 128), jnp.float32),
        scratch_types=[
            pltpu.VMEM((3, _RPAD), jnp.int32),
            pltpu.VMEM((_RPW, 128), jnp.float32),
            pltpu.VMEM((_RPW, 128), jnp.float32),
            pltpu.VMEM((_RPW, 128), jnp.float32),
            pltpu.SemaphoreType.DMA,
        ],
    )
    def k(t0, t1, t2, ridx_hbm, out_hbm, iv, s0, s1, s2, sem):
        wid = lax.axis_index("s") * 2 + lax.axis_index("c")
        base = wid * _RPW
        pltpu.sync_copy(ridx_hbm, iv)
        copies = []
        for s, (tbl, buf) in enumerate(((t0, s0), (t1, s1), (t2, s2))):
            isl = iv.at[s, pl.ds(base, _RPW)]
            copies.append(
                pltpu.async_copy(tbl.at[isl, pl.ds(0, 128)], buf, sem))
        for c in copies:
            c.wait()
        for s, buf in enumerate((s0, s1, s2)):
            pltpu.sync_copy(buf, out_hbm.at[s, pl.ds(base, _RPW)])

    return k(v0, v1, v2, ridx)


def _f_bce(x):
    return jnp.maximum(x, 0.0) + jnp.log1p(jnp.exp(-jnp.abs(x)))


def _sigmoid(x):
    return 1.0 / (1.0 + jnp.exp(-x))


def _atan_pos(x):
    inv = x > 1.0
    t = jnp.where(inv, 1.0 / jnp.maximum(x, 1e-30), x)
    t2 = t * t
    p = -0.01172120
    p = p * t2 + 0.05265332
    p = p * t2 - 0.11643287
    p = p * t2 + 0.19354346
    p = p * t2 - 0.33262347
    p = p * t2 + 0.99997726
    r = t * p
    return jnp.where(inv, (math.pi / 2) - r, r)


def _ciou_cols(bx, by, bw, bh, tx, ty, tw, th, eps=1e-7):
    b1x1 = bx - bw / 2; b1x2 = bx + bw / 2
    b1y1 = by - bh / 2; b1y2 = by + bh / 2
    b2x1 = tx - tw / 2; b2x2 = tx + tw / 2
    b2y1 = ty - th / 2; b2y2 = ty + th / 2
    inter = (jnp.maximum(jnp.minimum(b1x2, b2x2) - jnp.maximum(b1x1, b2x1), 0.0)
             * jnp.maximum(jnp.minimum(b1y2, b2y2) - jnp.maximum(b1y1, b2y1), 0.0))
    w1 = b1x2 - b1x1; h1 = b1y2 - b1y1 + eps
    w2 = b2x2 - b2x1; h2 = b2y2 - b2y1 + eps
    union = w1 * h1 + w2 * h2 - inter + eps
    iou = inter / union
    cw = jnp.maximum(b1x2, b2x2) - jnp.minimum(b1x1, b2x1)
    ch = jnp.maximum(b1y2, b2y2) - jnp.minimum(b1y1, b2y1)
    c2 = cw ** 2 + ch ** 2 + eps
    rho2 = ((b2x1 + b2x2 - b1x1 - b1x2) ** 2
            + (b2y1 + b2y2 - b1y1 - b1y2) ** 2) / 4
    v = 4.0 / math.pi ** 2 * (_atan_pos(w2 / h2) - _atan_pos(w1 / h1)) ** 2
    alpha = v / (v - iou + (1.0 + eps))
    return iou - (rho2 / c2 + v * alpha)


def _scale_terms(pp, tb, an, gif, gjf, tcls, kc, kr, hw):
    px = pp[:, 0:1]; py = pp[:, 1:2]
    pw = pp[:, 2:3]; ph = pp[:, 3:4]; pobj = pp[:, 4:5]
    xy_x = _sigmoid(px) * 2.0 - 0.5
    xy_y = _sigmoid(py) * 2.0 - 0.5
    wh_w = (_sigmoid(pw) * 2.0) ** 2 * an[:, 0:1]
    wh_h = (_sigmoid(ph) * 2.0) ** 2 * an[:, 1:2]
    fs = float(hw)
    tx = tb[:, 0:1] * fs - gif
    ty = tb[:, 1:2] * fs - gjf
    tw = tb[:, 2:3] * fs
    th = tb[:, 3:4] * fs
    iou = _ciou_cols(xy_x, xy_y, wh_w, wh_h, tx, ty, tw, th)
    box_loss = 1.0 - jnp.sum(iou) / float(_N)
    keq = kc == kr
    ncol = lax.broadcasted_iota(jnp.int32, (_N, _N), 0)
    mrow = lax.broadcasted_iota(jnp.int32, (_N, _N), 1)
    later = jnp.where(keq & (mrow > ncol), 1.0, 0.0)
    has_later = jnp.sum(later, axis=1, keepdims=True)
    keep = jnp.where(has_later > 0.0, 1.0, 0.0)
    corr = jnp.sum(keep * pobj * jnp.maximum(iou, 0.0))
    cl = pp[:, 5:85]
    sumf = jnp.sum(_f_bce(cl))
    cm = lax.broadcasted_iota(jnp.int32, (_N, _NCLS), 1)
    pick = jnp.sum(jnp.where(cm == tcls, cl, 0.0))
    cls_loss = (sumf - pick) / float(_N * _NCLS)
    return box_loss, corr, cls_loss


def _tc1_body(x0, x1, x2, out_ref):
    i = pl.program_id(0)
    sel_lo = jnp.where(
        lax.broadcasted_iota(jnp.int32, (128, 2), 0)
        == 4 + 85 * lax.broadcasted_iota(jnp.int32, (128, 2), 1), 1.0, 0.0)
    sel_hi = jnp.where(
        lax.broadcasted_iota(jnp.int32, (127, 1), 0) == 46, 1.0, 0.0)
    acc = jnp.zeros((1, 1), jnp.float32)
    dn = (((0,), (1,)), ((), ()))
    for s, x in enumerate((x0, x1, x2)):
        hw = _HW[s]
        norm = _BAL[s] / float(16 * 3 * hw * hw)
        cols_lo = lax.dot_general(sel_lo, x[:, 0:128], dn,
                                  preferred_element_type=jnp.float32)
        cols_hi = lax.dot_general(sel_hi, x[:, 128:255], dn,
                                  preferred_element_type=jnp.float32)
        acc = acc + norm * (jnp.sum(_f_bce(cols_lo))
                            + jnp.sum(_f_bce(cols_hi))).reshape(1, 1)

    @pl.when(i == 0)
    def _():
        out_ref[...] = acc

    @pl.when(i != 0)
    def _():
        out_ref[...] = out_ref[...] + acc


_RING = 16


def _tc2_body(bands_sm, va0, va1, va2, lo0, lo1, lo2,
              tb0, tb1, tb2, an0, an1, an2,
              a0, rm0, gi0, gj0, tc0, kc0, kr0,
              a1, rm1, gi1, gj1, tc1, kc1, kr1,
              a2, rm2, gi2, gj2, tc2, kc2, kr2, plane, out_ref,
              hi0, hi1, hi2, sems):
    vs = (va0, va1, va2)
    his = (hi0, hi1, hi2)

    def start(s, i):
        band = bands_sm[s, i]
        return pltpu.make_async_copy(
            vs[s].at[pl.ds(pl.multiple_of(band * 8, 8), 8), pl.ds(128, 127)],
            his[s].at[i], sems.at[s, lax.rem(i, _RING)])

    def body(i, carry):
        for t in range(2):
            j = 2 * i + t - _RING

            @pl.when((j >= 0) & (j < _N))
            def _():
                for s in range(3):
                    start(s, j).wait()

        for t in range(2):
            k = 2 * i + t

            @pl.when(k < _N)
            def _():
                for s in range(3):
                    start(s, k).start()
        return carry

    lax.fori_loop(0, (_N + _RING) // 2, body, 0)

    total = plane[...]
    per_scale = (
        (lo0, hi0, tb0, an0, a0, rm0, gi0, gj0, tc0, kc0, kr0),
        (lo1, hi1, tb1, an1, a1, rm1, gi1, gj1, tc1, kc1, kr1),
        (lo2, hi2, tb2, an2, a2, rm2, gi2, gj2, tc2, kc2, kr2),
    )
    for s in range(3):
        lo, hi, tb, an, a, rm, gi, gj, tc, kc, kr = per_scale[s]
        hw = _HW[s]
        rmask = jnp.where(
            lax.broadcasted_iota(jnp.int32, (_N, 8), 1) == rm[...],
            1.0, 0.0)
        hirows = jnp.sum(hi[...][0:_N] * rmask[:, :, None], axis=1)
        rows = jnp.concatenate([lo[...][0:_N], hirows], axis=1)
        av = a[...]
        pp = jnp.where(av == 0, rows[:, 0:85],
                       jnp.where(av == 1, rows[:, 85:170], rows[:, 170:255]))
        box_l, corr, cls_l = _scale_terms(
            pp, tb[...], an[...],
            gi[...].astype(jnp.float32), gj[...].astype(jnp.float32),
            tc[...], kc[...], kr[...], hw)
        norm = _BAL[s] / float(16 * 3 * hw * hw)
        total = total + (0.05 * box_l - norm * corr
                         + 0.5 * cls_l).reshape(1, 1)
    out_ref[...] = total


def kernel(pred0, pred1, pred2, tbox0, tbox1, tbox2, anch0, anch1, anch2,
           b0, a0, gj0, gi0, tcls0, b1, a1, gj1, gi1, tcls1,
           b2, a2, gj2, gi2, tcls2):
    v0 = pred0.transpose(2, 3, 0, 1).reshape(20 * 20 * 16, _NCH)
    v1 = pred1.transpose(0, 2, 3, 1).reshape(16 * 40 * 40, _NCH)
    v2 = pred2.transpose(0, 2, 3, 1).reshape(16 * 80 * 80, _NCH)

    bs = (b0, b1, b2); ans = (a0, a1, a2)
    gjs = (gj0, gj1, gj2); gis = (gi0, gi1, gi2)
    tcls = (tcls0, tcls1, tcls2)

    ridx, bidx, rmods, kcs, krs, acols, gifs, gjfs, tccols = (
        [], [], [], [], [], [], [], [], [])
    for s, hw in enumerate(_HW):
        b = bs[s].astype(jnp.int32)
        a = ans[s].astype(jnp.int32)
        gj = gjs[s].astype(jnp.int32)
        gi = gis[s].astype(jnp.int32)
        if s == 0:
            r = (gj * hw + gi) * 16 + b
        else:
            r = (b * hw + gj) * hw + gi
        pad = jnp.zeros((_RPAD - _N,), jnp.int32)
        ridx.append(jnp.concatenate([r, pad]))
        bidx.append(jnp.concatenate([r // 8, pad]))
        rmods.append((r % 8)[:, None])
        key = ((b * 3 + a) * hw + gj) * hw + gi
        kcs.append(key[:, None])
        krs.append(key[None, :])
        acols.append(a[:, None])
        gifs.append(gi[:, None])
        gjfs.append(gj[:, None])
        tccols.append(tcls[s][:, None].astype(jnp.int32))
    ridx_all = jnp.stack(ridx)
    bidx_all = jnp.stack(bidx)

    blk = lambda n: pl.BlockSpec((n, _NCH), lambda i: (i, 0))
    plane = pl.pallas_call(
        _tc1_body,
        grid=(_G1,),
        in_specs=[blk(6400 // _G1), blk(25600 // _G1), blk(102400 // _G1)],
        out_specs=pl.BlockSpec((1, 1), lambda i: (0, 0)),
        out_shape=jax.ShapeDtypeStruct((1, 1), jnp.float32),
    )(v0, v1, v2)

    lo = _sc_gather_lo(v0, v1, v2, ridx_all)

    full = lambda shape: pl.BlockSpec(shape, lambda: tuple(0 for _ in shape))
    out = pl.pallas_call(
        _tc2_body,
        in_specs=[
            pl.BlockSpec(memory_space=pltpu.MemorySpace.SMEM),
            pl.BlockSpec(memory_space=pl.ANY),
            pl.BlockSpec(memory_space=pl.ANY),
            pl.BlockSpec(memory_space=pl.ANY),
            full((_RPAD, 128)), full((_RPAD, 128)), full((_RPAD, 128)),
            full((_N, 4)), full((_N, 4)), full((_N, 4)),
            full((_N, 2)), full((_N, 2)), full((_N, 2)),
            full((_N, 1)), full((_N, 1)), full((_N, 1)), full((_N, 1)),
            full((_N, 1)), full((_N, 1)), full((1, _N)),
            full((_N, 1)), full((_N, 1)), full((_N, 1)), full((_N, 1)),
            full((_N, 1)), full((_N, 1)), full((1, _N)),
            full((_N, 1)), full((_N, 1)), full((_N, 1)), full((_N, 1)),
            full((_N, 1)), full((_N, 1)), full((1, _N)),
            full((1, 1)),
        ],
        out_specs=full((1, 1)),
        out_shape=jax.ShapeDtypeStruct((1, 1), jnp.float32),
        scratch_shapes=[
            pltpu.VMEM((_N, 8, 127), jnp.float32),
            pltpu.VMEM((_N, 8, 127), jnp.float32),
            pltpu.VMEM((_N, 8, 127), jnp.float32),
            pltpu.SemaphoreType.DMA((3, _RING)),
        ],
    )(bidx_all, v0, v1, v2, lo[0], lo[1], lo[2],
      tbox0, tbox1, tbox2, anch0, anch1, anch2,
      acols[0], rmods[0], gifs[0], gjfs[0], tccols[0], kcs[0], krs[0],
      acols[1], rmods[1], gifs[1], gjfs[1], tccols[1], kcs[1], krs[1],
      acols[2], rmods[2], gifs[2], gjfs[2], tccols[2], kcs[2], krs[2],
      plane)
    return out.reshape(1)


# speedup vs baseline: 8.7671x; 1.1439x over previous
import functools
import math

import jax
import jax.numpy as jnp
from jax import lax
from jax.experimental import pallas as pl
from jax.experimental.pallas import tpu as pltpu
from jax.experimental.pallas import tpu_sc as plsc

_HW = (20, 40, 80)
_N = 300
_NCH = 255
_NCLS = 80
_BAL = (0.4, 1.0, 4.0)
_NW = 32
_RPW = 16
_RPAD = _NW * _RPW
_G1 = 10


def _sc_gather_lo(v0, v1, v2, ridx):
    mesh = plsc.VectorSubcoreMesh(core_axis_name="c", subcore_axis_name="s")

    @functools.partial(
        pl.kernel,
        mesh=mesh,
        out_type=jax.ShapeDtypeStruct((3, _RPAD, 128), jnp.float32),
        scratch_types=[
            pltpu.VMEM((3, _RPAD), jnp.int32),
            pltpu.VMEM((_RPW, 128), jnp.float32),
            pltpu.VMEM((_RPW, 128), jnp.float32),
            pltpu.VMEM((_RPW, 128), jnp.float32),
            pltpu.SemaphoreType.DMA,
        ],
    )
    def k(t0, t1, t2, ridx_hbm, out_hbm, iv, s0, s1, s2, sem):
        wid = lax.axis_index("s") * 2 + lax.axis_index("c")
        base = wid * _RPW
        pltpu.sync_copy(ridx_hbm, iv)
        copies = []
        for s, (tbl, buf) in enumerate(((t0, s0), (t1, s1), (t2, s2))):
            isl = iv.at[s, pl.ds(base, _RPW)]
            copies.append(
                pltpu.async_copy(tbl.at[isl, pl.ds(0, 128)], buf, sem))
        for c in copies:
            c.wait()
        for s, buf in enumerate((s0, s1, s2)):
            pltpu.sync_copy(buf, out_hbm.at[s, pl.ds(base, _RPW)])

    return k(v0, v1, v2, ridx)


def _f_bce(x):
    return jnp.maximum(x, 0.0) + jnp.log1p(jnp.exp(-jnp.abs(x)))


def _sigmoid(x):
    return 1.0 / (1.0 + jnp.exp(-x))


def _atan_pos(x):
    inv = x > 1.0
    t = jnp.where(inv, 1.0 / jnp.maximum(x, 1e-30), x)
    t2 = t * t
    p = -0.01172120
    p = p * t2 + 0.05265332
    p = p * t2 - 0.11643287
    p = p * t2 + 0.19354346
    p = p * t2 - 0.33262347
    p = p * t2 + 0.99997726
    r = t * p
    return jnp.where(inv, (math.pi / 2) - r, r)


def _ciou_cols(bx, by, bw, bh, tx, ty, tw, th, eps=1e-7):
    b1x1 = bx - bw / 2; b1x2 = bx + bw / 2
    b1y1 = by - bh / 2; b1y2 = by + bh / 2
    b2x1 = tx - tw / 2; b2x2 = tx + tw / 2
    b2y1 = ty - th / 2; b2y2 = ty + th / 2
    inter = (jnp.maximum(jnp.minimum(b1x2, b2x2) - jnp.maximum(b1x1, b2x1), 0.0)
             * jnp.maximum(jnp.minimum(b1y2, b2y2) - jnp.maximum(b1y1, b2y1), 0.0))
    w1 = b1x2 - b1x1; h1 = b1y2 - b1y1 + eps
    w2 = b2x2 - b2x1; h2 = b2y2 - b2y1 + eps
    union = w1 * h1 + w2 * h2 - inter + eps
    iou = inter / union
    cw = jnp.maximum(b1x2, b2x2) - jnp.minimum(b1x1, b2x1)
    ch = jnp.maximum(b1y2, b2y2) - jnp.minimum(b1y1, b2y1)
    c2 = cw ** 2 + ch ** 2 + eps
    rho2 = ((b2x1 + b2x2 - b1x1 - b1x2) ** 2
            + (b2y1 + b2y2 - b1y1 - b1y2) ** 2) / 4
    v = 4.0 / math.pi ** 2 * (_atan_pos(w2 / h2) - _atan_pos(w1 / h1)) ** 2
    alpha = v / (v - iou + (1.0 + eps))
    return iou - (rho2 / c2 + v * alpha)


def _scale_terms(pp, tb, an, gif, gjf, tcls, kc, kr, hw):
    px = pp[:, 0:1]; py = pp[:, 1:2]
    pw = pp[:, 2:3]; ph = pp[:, 3:4]; pobj = pp[:, 4:5]
    xy_x = _sigmoid(px) * 2.0 - 0.5
    xy_y = _sigmoid(py) * 2.0 - 0.5
    wh_w = (_sigmoid(pw) * 2.0) ** 2 * an[:, 0:1]
    wh_h = (_sigmoid(ph) * 2.0) ** 2 * an[:, 1:2]
    fs = float(hw)
    tx = tb[:, 0:1] * fs - gif
    ty = tb[:, 1:2] * fs - gjf
    tw = tb[:, 2:3] * fs
    th = tb[:, 3:4] * fs
    iou = _ciou_cols(xy_x, xy_y, wh_w, wh_h, tx, ty, tw, th)
    box_loss = 1.0 - jnp.sum(iou) / float(_N)
    keq = kc == kr
    ncol = lax.broadcasted_iota(jnp.int32, (_N, _N), 0)
    mrow = lax.broadcasted_iota(jnp.int32, (_N, _N), 1)
    later = jnp.where(keq & (mrow > ncol), 1.0, 0.0)
    has_later = jnp.sum(later, axis=1, keepdims=True)
    keep = jnp.where(has_later > 0.0, 1.0, 0.0)
    corr = jnp.sum(keep * pobj * jnp.maximum(iou, 0.0))
    cl = pp[:, 5:85]
    sumf = jnp.sum(_f_bce(cl))
    cm = lax.broadcasted_iota(jnp.int32, (_N, _NCLS), 1)
    pick = jnp.sum(jnp.where(cm == tcls, cl, 0.0))
    cls_loss = (sumf - pick) / float(_N * _NCLS)
    return box_loss, corr, cls_loss


_RING = 16


def _tc_body(bands_sm, va0, va1, va2, x0, x1, x2, lo0, lo1, lo2,
             tb0, tb1, tb2, an0, an1, an2,
             a0, rm0, gi0, gj0, tc0, kc0, kr0,
             a1, rm1, gi1, gj1, tc1, kc1, kr1,
             a2, rm2, gi2, gj2, tc2, kc2, kr2, out_ref,
             hi0, hi1, hi2, sems):
    i = pl.program_id(0)
    vs = (va0, va1, va2)
    his = (hi0, hi1, hi2)

    sel_lo = jnp.where(
        lax.broadcasted_iota(jnp.int32, (128, 2), 0)
        == 4 + 85 * lax.broadcasted_iota(jnp.int32, (128, 2), 1), 1.0, 0.0)
    sel_hi = jnp.where(
        lax.broadcasted_iota(jnp.int32, (127, 1), 0) == 46, 1.0, 0.0)
    acc = jnp.zeros((1, 1), jnp.float32)
    dn = (((0,), (1,)), ((), ()))
    for s, x in enumerate((x0, x1, x2)):
        hw = _HW[s]
        norm = _BAL[s] / float(16 * 3 * hw * hw)
        cols_lo = lax.dot_general(sel_lo, x[:, 0:128], dn,
                                  preferred_element_type=jnp.float32)
        cols_hi = lax.dot_general(sel_hi, x[:, 128:255], dn,
                                  preferred_element_type=jnp.float32)
        acc = acc + norm * (jnp.sum(_f_bce(cols_lo))
                            + jnp.sum(_f_bce(cols_hi))).reshape(1, 1)

    def start(s, n):
        band = bands_sm[s, n]
        return pltpu.make_async_copy(
            vs[s].at[pl.ds(pl.multiple_of(band * 8, 8), 8), pl.ds(128, 127)],
            his[s].at[n], sems.at[s, lax.rem(n, _RING)])

    @pl.when(i == 0)
    def _():
        out_ref[...] = acc

        def body(it, carry):
            for t in range(2):
                j = 2 * it + t - _RING

                @pl.when((j >= 0) & (j < _N))
                def _():
                    for s in range(3):
                        start(s, j).wait()

            for t in range(2):
                k = 2 * it + t

                @pl.when(k < _N)
                def _():
                    for s in range(3):
                        start(s, k).start()
            return carry

        lax.fori_loop(0, (_N + _RING) // 2, body, 0)

    @pl.when(i != 0)
    def _():
        out_ref[...] = out_ref[...] + acc

    @pl.when(i == _G1 - 1)
    def _():
        extra = jnp.zeros((1, 1), jnp.float32)
        per_scale = (
            (lo0, hi0, tb0, an0, a0, rm0, gi0, gj0, tc0, kc0, kr0),
            (lo1, hi1, tb1, an1, a1, rm1, gi1, gj1, tc1, kc1, kr1),
            (lo2, hi2, tb2, an2, a2, rm2, gi2, gj2, tc2, kc2, kr2),
        )
        for s in range(3):
            lo, hi, tb, an, a, rm, gi, gj, tc, kc, kr = per_scale[s]
            hw = _HW[s]
            rmask = jnp.where(
                lax.broadcasted_iota(jnp.int32, (_N, 8), 1) == rm[...],
                1.0, 0.0)
            hirows = jnp.sum(hi[...][0:_N] * rmask[:, :, None], axis=1)
            rows = jnp.concatenate([lo[...][0:_N], hirows], axis=1)
            av = a[...]
            pp = jnp.where(
                av == 0, rows[:, 0:85],
                jnp.where(av == 1, rows[:, 85:170], rows[:, 170:255]))
            box_l, corr, cls_l = _scale_terms(
                pp, tb[...], an[...],
                gi[...].astype(jnp.float32), gj[...].astype(jnp.float32),
                tc[...], kc[...], kr[...], hw)
            norm = _BAL[s] / float(16 * 3 * hw * hw)
            extra = extra + (0.05 * box_l - norm * corr
                             + 0.5 * cls_l).reshape(1, 1)
        out_ref[...] = out_ref[...] + extra


def kernel(pred0, pred1, pred2, tbox0, tbox1, tbox2, anch0, anch1, anch2,
           b0, a0, gj0, gi0, tcls0, b1, a1, gj1, gi1, tcls1,
           b2, a2, gj2, gi2, tcls2):
    v0 = pred0.transpose(2, 3, 0, 1).reshape(20 * 20 * 16, _NCH)
    v1 = pred1.transpose(0, 2, 3, 1).reshape(16 * 40 * 40, _NCH)
    v2 = pred2.transpose(0, 2, 3, 1).reshape(16 * 80 * 80, _NCH)

    bs = (b0, b1, b2); ans = (a0, a1, a2)
    gjs = (gj0, gj1, gj2); gis = (gi0, gi1, gi2)
    tcls = (tcls0, tcls1, tcls2)

    ridx, bidx, rmods, kcs, krs, acols, gifs, gjfs, tccols = (
        [], [], [], [], [], [], [], [], [])
    for s, hw in enumerate(_HW):
        b = bs[s].astype(jnp.int32)
        a = ans[s].astype(jnp.int32)
        gj = gjs[s].astype(jnp.int32)
        gi = gis[s].astype(jnp.int32)
        if s == 0:
            r = (gj * hw + gi) * 16 + b
        else:
            r = (b * hw + gj) * hw + gi
        pad = jnp.zeros((_RPAD - _N,), jnp.int32)
        ridx.append(jnp.concatenate([r, pad]))
        bidx.append(jnp.concatenate([r // 8, pad]))
        rmods.append((r % 8)[:, None])
        key = ((b * 3 + a) * hw + gj) * hw + gi
        kcs.append(key[:, None])
        krs.append(key[None, :])
        acols.append(a[:, None])
        gifs.append(gi[:, None])
        gjfs.append(gj[:, None])
        tccols.append(tcls[s][:, None].astype(jnp.int32))
    ridx_all = jnp.stack(ridx)
    bidx_all = jnp.stack(bidx)

    blk = lambda n: pl.BlockSpec((n, _NCH), lambda i: (i, 0))
    fullg = lambda shape: pl.BlockSpec(shape, lambda i: tuple(0 for _ in shape))

    lo = _sc_gather_lo(v0, v1, v2, ridx_all)

    out = pl.pallas_call(
        _tc_body,
        grid=(_G1,),
        in_specs=[
            pl.BlockSpec(memory_space=pltpu.MemorySpace.SMEM),
            pl.BlockSpec(memory_space=pl.ANY),
            pl.BlockSpec(memory_space=pl.ANY),
            pl.BlockSpec(memory_space=pl.ANY),
            blk(6400 // _G1), blk(25600 // _G1), blk(102400 // _G1),
            fullg((_RPAD, 128)), fullg((_RPAD, 128)), fullg((_RPAD, 128)),
            fullg((_N, 4)), fullg((_N, 4)), fullg((_N, 4)),
            fullg((_N, 2)), fullg((_N, 2)), fullg((_N, 2)),
            fullg((_N, 1)), fullg((_N, 1)), fullg((_N, 1)), fullg((_N, 1)),
            fullg((_N, 1)), fullg((_N, 1)), fullg((1, _N)),
            fullg((_N, 1)), fullg((_N, 1)), fullg((_N, 1)), fullg((_N, 1)),
            fullg((_N, 1)), fullg((_N, 1)), fullg((1, _N)),
            fullg((_N, 1)), fullg((_N, 1)), fullg((_N, 1)), fullg((_N, 1)),
            fullg((_N, 1)), fullg((_N, 1)), fullg((1, _N)),
        ],
        out_specs=fullg((1, 1)),
        out_shape=jax.ShapeDtypeStruct((1, 1), jnp.float32),
        scratch_shapes=[
            pltpu.VMEM((_N, 8, 127), jnp.float32),
            pltpu.VMEM((_N, 8, 127), jnp.float32),
            pltpu.VMEM((_N, 8, 127), jnp.float32),
            pltpu.SemaphoreType.DMA((3, _RING)),
        ],
    )(bidx_all, v0, v1, v2, v0, v1, v2, lo[0], lo[1], lo[2],
      tbox0, tbox1, tbox2, anch0, anch1, anch2,
      acols[0], rmods[0], gifs[0], gjfs[0], tccols[0], kcs[0], krs[0],
      acols[1], rmods[1], gifs[1], gjfs[1], tccols[1], kcs[1], krs[1],
      acols[2], rmods[2], gifs[2], gjfs[2], tccols[2], kcs[2], krs[2])
    return out.reshape(1)
